# baseline (speedup 1.0000x reference)
import functools

import jax
import jax.numpy as jnp
from jax import lax
from jax.experimental import pallas as pl
from jax.experimental.pallas import tpu as pltpu
from jax.experimental.pallas import tpu_sc as plsc

N_NODES = 10000
N_EDGES = 320000
D_IN = 128
D_HID = 64
D_OUT = 2

NC = 2
NS = 16
NW = NC * NS
CHUNK = 128
CHUNKS_PER_TILE = 80
EDGES_PER_TILE = CHUNKS_PER_TILE * CHUNK
E_PAD = EDGES_PER_TILE * NW
IDX_ROWS = CHUNKS_PER_TILE + 2
N_PAD = 10112
ROWS_PER_TILE = N_PAD // NS
CW = 16
ROW_BLK = 1000


def _sc_scatter(with_cnt):
    mesh = plsc.VectorSubcoreMesh(core_axis_name="c", subcore_axis_name="s")
    agg_t = jax.ShapeDtypeStruct((NC, N_PAD, D_HID), jnp.float32)
    out_type = [agg_t] if with_cnt else agg_t
    scratch = [
        pltpu.VMEM((IDX_ROWS, CHUNK), jnp.int32),
        pltpu.VMEM((IDX_ROWS, CHUNK), jnp.int32),
        pltpu.VMEM((CHUNK, D_HID), jnp.float32),
        pltpu.VMEM((CHUNK, D_HID), jnp.float32),
        pltpu.VMEM_SHARED((N_PAD, D_HID), jnp.float32),
        pltpu.SemaphoreType.DMA,
        pltpu.SemaphoreType.DMA,
    ]
    if with_cnt:
        out_type.append(jax.ShapeDtypeStruct((NC, N_PAD, CW), jnp.float32))
        scratch += [
            pltpu.VMEM((CHUNK, CW), jnp.float32),
            pltpu.VMEM_SHARED((N_PAD, CW), jnp.float32),
            pltpu.SemaphoreType.DMA,
            pltpu.SemaphoreType.DMA,
        ]

    def body(*refs):
        if with_cnt:
            (p, src3, dst3, ones_h, z64, z16, agg_o, cnt_o,
             sv, dv, rA, rB, acc, semA, semB,
             ones_v, cacc, semCA, semCB) = refs
        else:
            (p, src3, dst3, z64, agg_o,
             sv, dv, rA, rB, acc, semA, semB) = refs

        cid = lax.axis_index("c")
        sid = lax.axis_index("s")
        wid = cid * NS + sid
        r0 = sid * ROWS_PER_TILE

        pltpu.sync_copy(z64, acc.at[pl.ds(r0, ROWS_PER_TILE)])
        if with_cnt:
            pltpu.sync_copy(z16, cacc.at[pl.ds(r0, ROWS_PER_TILE)])
            pltpu.sync_copy(ones_h, ones_v)
        pltpu.sync_copy(src3.at[wid], sv)
        pltpu.sync_copy(dst3.at[wid], dv)
        plsc.subcore_barrier()

        def g_start(j, rv, sem):
            pltpu.async_copy(p.at[sv.at[j]], rv, sem)

        def g_drain(j, rv, sem):
            pltpu.make_async_copy(p.at[sv.at[j]], rv, sem).wait()

        def r_scat(j, rv):
            pltpu.sync_copy(rv, acc.at[dv.at[j]], add=True)

        def c_fire(j, sem):
            pltpu.async_copy(ones_v, cacc.at[dv.at[j]], sem, add=True)

        def c_wait(j, sem):
            pltpu.make_async_copy(ones_v, cacc.at[dv.at[j]], sem).wait()

        g_start(0, rA, semA)
        if with_cnt:
            c_fire(0, semCA)
            c_fire(1, semCB)

        @pl.loop(0, CHUNKS_PER_TILE // 2)
        def _(it):
            jA = it * 2
            jB = jA + 1
            g_start(jB, rB, semB)
            g_drain(jA, rA, semA)
            r_scat(jA, rA)
            if with_cnt:
                c_wait(jA, semCA)
                c_fire(jA + 2, semCA)
            g_start(jA + 2, rA, semA)
            g_drain(jB, rB, semB)
            r_scat(jB, rB)
            if with_cnt:
                c_wait(jB, semCB)
                c_fire(jB + 2, semCB)

        g_drain(CHUNKS_PER_TILE, rA, semA)
        if with_cnt:
            c_wait(CHUNKS_PER_TILE, semCA)
            c_wait(CHUNKS_PER_TILE + 1, semCB)

        plsc.subcore_barrier()
        pltpu.sync_copy(acc.at[pl.ds(r0, ROWS_PER_TILE)],
                        agg_o.at[cid].at[pl.ds(r0, ROWS_PER_TILE)])
        if with_cnt:
            pltpu.sync_copy(cacc.at[pl.ds(r0, ROWS_PER_TILE)],
                            cnt_o.at[cid].at[pl.ds(r0, ROWS_PER_TILE)])

    cp = pltpu.CompilerParams(use_tc_tiling_on_sc=False)
    return pl.kernel(body, out_type=out_type, mesh=mesh, scratch_types=scratch,
                     compiler_params=cp)


def _dense2(x, Wl, Wr, b2d):
    def tc_body(x_ref, wl_ref, wr_ref, b_ref, p_ref, q_ref):
        xb = x_ref[...]
        p_ref[...] = jnp.dot(xb, wl_ref[...],
                             preferred_element_type=jnp.float32,
                             precision=lax.Precision.HIGHEST)
        q_ref[...] = jnp.dot(xb, wr_ref[...],
                             preferred_element_type=jnp.float32,
                             precision=lax.Precision.HIGHEST) + b_ref[...]

    return pl.pallas_call(
        tc_body,
        grid=(N_NODES // ROW_BLK,),
        in_specs=[pl.BlockSpec((ROW_BLK, D_IN), lambda i: (i, 0)),
                  pl.BlockSpec((D_IN, D_HID), lambda i: (0, 0)),
                  pl.BlockSpec((D_IN, D_HID), lambda i: (0, 0)),
                  pl.BlockSpec((1, D_HID), lambda i: (0, 0))],
        out_specs=[pl.BlockSpec((ROW_BLK, D_HID), lambda i: (i, 0)),
                   pl.BlockSpec((ROW_BLK, D_HID), lambda i: (i, 0))],
        out_shape=[jax.ShapeDtypeStruct((N_NODES, D_HID), jnp.float32)] * 2,
    )(x, Wl, Wr, b2d)


def _mid(aggp, cntp, Q1, W2l, W2r, b2d):
    def tc_body(a_ref, c_ref, q_ref, wl_ref, wr_ref, b_ref, p_ref, q2_ref):
        a = a_ref[0] + a_ref[1]
        cnt = c_ref[0, :, 0:1] + c_ref[1, :, 0:1]
        inv = 1.0 / jnp.maximum(cnt, 1.0)
        h = jnp.maximum(a * inv + q_ref[...], 0.0)
        p_ref[...] = jnp.dot(h, wl_ref[...],
                             preferred_element_type=jnp.float32,
                             precision=lax.Precision.HIGHEST)
        q2_ref[...] = jnp.dot(h, wr_ref[...],
                              preferred_element_type=jnp.float32,
                              precision=lax.Precision.HIGHEST) + b_ref[...]

    return pl.pallas_call(
        tc_body,
        grid=(N_NODES // ROW_BLK,),
        in_specs=[pl.BlockSpec((NC, ROW_BLK, D_HID), lambda i: (0, i, 0)),
                  pl.BlockSpec((NC, ROW_BLK, CW), lambda i: (0, i, 0)),
                  pl.BlockSpec((ROW_BLK, D_HID), lambda i: (i, 0)),
                  pl.BlockSpec((D_HID, D_HID), lambda i: (0, 0)),
                  pl.BlockSpec((D_HID, D_HID), lambda i: (0, 0)),
                  pl.BlockSpec((1, D_HID), lambda i: (0, 0))],
        out_specs=[pl.BlockSpec((ROW_BLK, D_HID), lambda i: (i, 0)),
                   pl.BlockSpec((ROW_BLK, D_HID), lambda i: (i, 0))],
        out_shape=[jax.ShapeDtypeStruct((N_NODES, D_HID), jnp.float32)] * 2,
    )(aggp, cntp, Q1, W2l, W2r, b2d)


def _final(aggp, cntp, Q2, Wpad, bpad):
    def tc_body(a_ref, c_ref, q_ref, w_ref, b_ref, o_ref):
        a = a_ref[0] + a_ref[1]
        cnt = c_ref[0, :, 0:1] + c_ref[1, :, 0:1]
        inv = 1.0 / jnp.maximum(cnt, 1.0)
        h = jnp.maximum(a * inv + q_ref[...], 0.0)
        o_ref[...] = jnp.dot(h, w_ref[...],
                             preferred_element_type=jnp.float32,
                             precision=lax.Precision.HIGHEST) + b_ref[...]

    return pl.pallas_call(
        tc_body,
        grid=(N_NODES // ROW_BLK,),
        in_specs=[pl.BlockSpec((NC, ROW_BLK, D_HID), lambda i: (0, i, 0)),
                  pl.BlockSpec((NC, ROW_BLK, CW), lambda i: (0, i, 0)),
                  pl.BlockSpec((ROW_BLK, D_HID), lambda i: (i, 0)),
                  pl.BlockSpec((D_HID, 128), lambda i: (0, 0)),
                  pl.BlockSpec((1, 128), lambda i: (0, 0))],
        out_specs=pl.BlockSpec((ROW_BLK, 128), lambda i: (i, 0)),
        out_shape=jax.ShapeDtypeStruct((N_NODES, 128), jnp.float32),
    )(aggp, cntp, Q2, Wpad, bpad)


def _pad_rows(a):
    return jnp.concatenate(
        [a, jnp.zeros((N_PAD - N_NODES, a.shape[1]), a.dtype)])


def kernel(x, edge_index, W1l, b1l, W1r, b1r, W2l, b2l, W2r, b2r, Wlin, blin):
    f32 = jnp.float32
    src = edge_index[0].astype(jnp.int32)
    dst = edge_index[1].astype(jnp.int32)
    npad = E_PAD - N_EDGES

    def idx3(v):
        vp = jnp.concatenate([v, jnp.full((npad,), N_NODES, jnp.int32)])
        vp = vp.reshape(NW, CHUNKS_PER_TILE, CHUNK)
        tail = jnp.full((NW, IDX_ROWS - CHUNKS_PER_TILE, CHUNK),
                        N_NODES, jnp.int32)
        return jnp.concatenate([vp, tail], axis=1)

    src3 = idx3(src)
    dst3 = idx3(dst)
    ones = jnp.ones((CHUNK, CW), f32)
    z64 = jnp.zeros((ROWS_PER_TILE, D_HID), f32)
    z16 = jnp.zeros((ROWS_PER_TILE, CW), f32)

    P1, Q1 = _dense2(x, W1l, W1r, (b1l + b1r).reshape(1, -1))
    agg1, cntp = _sc_scatter(True)(_pad_rows(P1), src3, dst3, ones, z64, z16)
    P2, Q2 = _mid(agg1, cntp, Q1, W2l, W2r, (b2l + b2r).reshape(1, -1))
    agg2 = _sc_scatter(False)(_pad_rows(P2), src3, dst3, z64)
    Wpad = jnp.pad(Wlin, ((0, 0), (0, 128 - D_OUT)))
    bpad = jnp.pad(blin, (0, 128 - D_OUT)).reshape(1, -1)
    outp = _final(agg2, cntp, Q2, Wpad, bpad)
    return outp[:, :D_OUT]


# speedup vs baseline: 3.1022x; 3.1022x over previous
import functools

import jax
import jax.numpy as jnp
from jax import lax
from jax.experimental import pallas as pl
from jax.experimental.pallas import tpu as pltpu
from jax.experimental.pallas import tpu_sc as plsc

N_NODES = 10000
N_EDGES = 320000
D_IN = 128
D_HID = 64
D_OUT = 2

NC = 2
NS = 16
NW = NC * NS
CHUNK = 128
CHUNKS_PER_TILE = 80
EDGES_PER_TILE = CHUNKS_PER_TILE * CHUNK
E_PAD = EDGES_PER_TILE * NW
IDX_ROWS = CHUNKS_PER_TILE + 2
N_PAD = 10112
ROWS_PER_TILE = N_PAD // NS
CW = 16
ROW_BLK = 1000


def _sc_scatter(with_cnt):
    mesh = plsc.VectorSubcoreMesh(core_axis_name="c", subcore_axis_name="s")
    agg_t = jax.ShapeDtypeStruct((NC, N_PAD, D_HID), jnp.float32)
    out_type = [agg_t] if with_cnt else agg_t
    scratch = [
        pltpu.VMEM((IDX_ROWS, CHUNK), jnp.int32),
        pltpu.VMEM((IDX_ROWS, CHUNK), jnp.int32),
        pltpu.VMEM((CHUNK, D_HID), jnp.float32),
        pltpu.VMEM((CHUNK, D_HID), jnp.float32),
        pltpu.VMEM_SHARED((N_PAD, D_HID), jnp.float32),
        pltpu.SemaphoreType.DMA,
        pltpu.SemaphoreType.DMA,
    ]
    if with_cnt:
        out_type.append(jax.ShapeDtypeStruct((NC, N_PAD, CW), jnp.float32))
        scratch += [
            pltpu.VMEM((CHUNK, CW), jnp.float32),
            pltpu.VMEM_SHARED((N_PAD, CW), jnp.float32),
            pltpu.SemaphoreType.DMA,
            pltpu.SemaphoreType.DMA,
        ]

    def body(*refs):
        if with_cnt:
            (p, src3, dst3, ones_h, z64, z16, agg_o, cnt_o,
             sv, dv, rA, rB, acc, semA, semB,
             ones_v, cacc, semCA, semCB) = refs
        else:
            (p, src3, dst3, z64, agg_o,
             sv, dv, rA, rB, acc, semA, semB) = refs

        cid = lax.axis_index("c")
        sid = lax.axis_index("s")
        wid = cid * NS + sid
        r0 = sid * ROWS_PER_TILE

        pltpu.sync_copy(z64, acc.at[pl.ds(r0, ROWS_PER_TILE)])
        if with_cnt:
            pltpu.sync_copy(z16, cacc.at[pl.ds(r0, ROWS_PER_TILE)])
            pltpu.sync_copy(ones_h, ones_v)
        pltpu.sync_copy(src3.at[wid], sv)
        pltpu.sync_copy(dst3.at[wid], dv)
        plsc.subcore_barrier()

        def g_start(j, rv, sem):
            pltpu.async_copy(p.at[sv.at[j]], rv, sem)

        def g_drain(j, rv, sem):
            pltpu.make_async_copy(p.at[sv.at[j]], rv, sem).wait()

        def r_scat(j, rv):
            pltpu.sync_copy(rv, acc.at[dv.at[j]], add=True)

        def c_fire(j, sem):
            pltpu.async_copy(ones_v, cacc.at[dv.at[j]], sem, add=True)

        def c_wait(j, sem):
            pltpu.make_async_copy(ones_v, cacc.at[dv.at[j]], sem).wait()

        g_start(0, rA, semA)
        if with_cnt:
            c_fire(0, semCA)
            c_fire(1, semCB)

        @pl.loop(0, CHUNKS_PER_TILE // 2)
        def _(it):
            jA = it * 2
            jB = jA + 1
            g_start(jB, rB, semB)
            g_drain(jA, rA, semA)
            r_scat(jA, rA)
            if with_cnt:
                c_wait(jA, semCA)
                c_fire(jA + 2, semCA)
            g_start(jA + 2, rA, semA)
            g_drain(jB, rB, semB)
            r_scat(jB, rB)
            if with_cnt:
                c_wait(jB, semCB)
                c_fire(jB + 2, semCB)

        g_drain(CHUNKS_PER_TILE, rA, semA)
        if with_cnt:
            c_wait(CHUNKS_PER_TILE, semCA)
            c_wait(CHUNKS_PER_TILE + 1, semCB)

        plsc.subcore_barrier()
        pltpu.sync_copy(acc.at[pl.ds(r0, ROWS_PER_TILE)],
                        agg_o.at[cid].at[pl.ds(r0, ROWS_PER_TILE)])
        if with_cnt:
            pltpu.sync_copy(cacc.at[pl.ds(r0, ROWS_PER_TILE)],
                            cnt_o.at[cid].at[pl.ds(r0, ROWS_PER_TILE)])

    cp = pltpu.CompilerParams(use_tc_tiling_on_sc=False)
    return pl.kernel(body, out_type=out_type, mesh=mesh, scratch_types=scratch,
                     compiler_params=cp)


def _dense2(x, Wl, Wr, b2d):
    def tc_body(x_ref, wl_ref, wr_ref, b_ref, p_ref, q_ref):
        xb = x_ref[...]
        p_ref[...] = jnp.dot(xb, wl_ref[...],
                             preferred_element_type=jnp.float32,
                             precision=lax.Precision.HIGHEST)
        q_ref[...] = jnp.dot(xb, wr_ref[...],
                             preferred_element_type=jnp.float32,
                             precision=lax.Precision.HIGHEST) + b_ref[...]

    return pl.pallas_call(
        tc_body,
        grid=(N_NODES // ROW_BLK,),
        in_specs=[pl.BlockSpec((ROW_BLK, D_IN), lambda i: (i, 0)),
                  pl.BlockSpec((D_IN, D_HID), lambda i: (0, 0)),
                  pl.BlockSpec((D_IN, D_HID), lambda i: (0, 0)),
                  pl.BlockSpec((1, D_HID), lambda i: (0, 0))],
        out_specs=[pl.BlockSpec((ROW_BLK, D_HID), lambda i: (i, 0)),
                   pl.BlockSpec((ROW_BLK, D_HID), lambda i: (i, 0))],
        out_shape=[jax.ShapeDtypeStruct((N_NODES, D_HID), jnp.float32)] * 2,
    )(x, Wl, Wr, b2d)


def _mid(aggp, cntp, Q1, W2l, W2r, b2d):
    def tc_body(a_ref, c_ref, q_ref, wl_ref, wr_ref, b_ref, p_ref, q2_ref):
        a = a_ref[0] + a_ref[1]
        cnt = c_ref[0, :, 0:1] + c_ref[1, :, 0:1]
        inv = 1.0 / jnp.maximum(cnt, 1.0)
        h = jnp.maximum(a * inv + q_ref[...], 0.0)
        p_ref[...] = jnp.dot(h, wl_ref[...],
                             preferred_element_type=jnp.float32,
                             precision=lax.Precision.HIGHEST)
        q2_ref[...] = jnp.dot(h, wr_ref[...],
                              preferred_element_type=jnp.float32,
                              precision=lax.Precision.HIGHEST) + b_ref[...]

    return pl.pallas_call(
        tc_body,
        grid=(N_NODES // ROW_BLK,),
        in_specs=[pl.BlockSpec((NC, ROW_BLK, D_HID), lambda i: (0, i, 0)),
                  pl.BlockSpec((NC, ROW_BLK, CW), lambda i: (0, i, 0)),
                  pl.BlockSpec((ROW_BLK, D_HID), lambda i: (i, 0)),
                  pl.BlockSpec((D_HID, D_HID), lambda i: (0, 0)),
                  pl.BlockSpec((D_HID, D_HID), lambda i: (0, 0)),
                  pl.BlockSpec((1, D_HID), lambda i: (0, 0))],
        out_specs=[pl.BlockSpec((ROW_BLK, D_HID), lambda i: (i, 0)),
                   pl.BlockSpec((ROW_BLK, D_HID), lambda i: (i, 0))],
        out_shape=[jax.ShapeDtypeStruct((N_NODES, D_HID), jnp.float32)] * 2,
    )(aggp, cntp, Q1, W2l, W2r, b2d)


def _final(aggp, cntp, Q2, Wpad, bpad):
    def tc_body(a_ref, c_ref, q_ref, w_ref, b_ref, o_ref):
        a = a_ref[0] + a_ref[1]
        cnt = c_ref[0, :, 0:1] + c_ref[1, :, 0:1]
        inv = 1.0 / jnp.maximum(cnt, 1.0)
        h = jnp.maximum(a * inv + q_ref[...], 0.0)
        o_ref[...] = jnp.dot(h, w_ref[...],
                             preferred_element_type=jnp.float32,
                             precision=lax.Precision.HIGHEST) + b_ref[...]

    return pl.pallas_call(
        tc_body,
        grid=(N_NODES // ROW_BLK,),
        in_specs=[pl.BlockSpec((NC, ROW_BLK, D_HID), lambda i: (0, i, 0)),
                  pl.BlockSpec((NC, ROW_BLK, CW), lambda i: (0, i, 0)),
                  pl.BlockSpec((ROW_BLK, D_HID), lambda i: (i, 0)),
                  pl.BlockSpec((D_HID, 128), lambda i: (0, 0)),
                  pl.BlockSpec((1, 128), lambda i: (0, 0))],
        out_specs=pl.BlockSpec((ROW_BLK, 128), lambda i: (i, 0)),
        out_shape=jax.ShapeDtypeStruct((N_NODES, 128), jnp.float32),
    )(aggp, cntp, Q2, Wpad, bpad)


def _pad_rows(a):
    return jnp.concatenate(
        [a, jnp.zeros((N_PAD - N_NODES, a.shape[1]), a.dtype)])


def kernel(x, edge_index, W1l, b1l, W1r, b1r, W2l, b2l, W2r, b2r, Wlin, blin):
    f32 = jnp.float32
    src = edge_index[0].astype(jnp.int32)
    dst = edge_index[1].astype(jnp.int32)
    npad = E_PAD - N_EDGES

    def _padv(n):
        return (N_NODES
                + jnp.arange(n, dtype=jnp.int32) % (N_PAD - N_NODES))

    def idx3(v):
        vp = jnp.concatenate([v, _padv(npad)])
        vp = vp.reshape(NW, CHUNKS_PER_TILE, CHUNK)
        ntail = NW * (IDX_ROWS - CHUNKS_PER_TILE) * CHUNK
        tail = _padv(ntail).reshape(NW, IDX_ROWS - CHUNKS_PER_TILE, CHUNK)
        return jnp.concatenate([vp, tail], axis=1)

    src3 = idx3(src)
    dst3 = idx3(dst)
    ones = jnp.ones((CHUNK, CW), f32)
    z64 = jnp.zeros((ROWS_PER_TILE, D_HID), f32)
    z16 = jnp.zeros((ROWS_PER_TILE, CW), f32)

    P1, Q1 = _dense2(x, W1l, W1r, (b1l + b1r).reshape(1, -1))
    agg1, cntp = _sc_scatter(True)(_pad_rows(P1), src3, dst3, ones, z64, z16)
    P2, Q2 = _mid(agg1, cntp, Q1, W2l, W2r, (b2l + b2r).reshape(1, -1))
    agg2 = _sc_scatter(False)(_pad_rows(P2), src3, dst3, z64)
    Wpad = jnp.pad(Wlin, ((0, 0), (0, 128 - D_OUT)))
    bpad = jnp.pad(blin, (0, 128 - D_OUT)).reshape(1, -1)
    outp = _final(agg2, cntp, Q2, Wpad, bpad)
    return outp[:, :D_OUT]


# speedup vs baseline: 3.4965x; 1.1271x over previous
import functools

import jax
import jax.numpy as jnp
from jax import lax
from jax.experimental import pallas as pl
from jax.experimental.pallas import tpu as pltpu
from jax.experimental.pallas import tpu_sc as plsc

N_NODES = 10000
N_EDGES = 320000
D_IN = 128
D_HID = 64
D_OUT = 2

NC = 2
NS = 16
NW = NC * NS
CHUNK = 128
EDGES_PER_TILE = N_EDGES // NW
FULL_CHUNKS = EDGES_PER_TILE // CHUNK
TAIL = EDGES_PER_TILE - FULL_CHUNKS * CHUNK
N_PAD = 10112
ROWS_PER_TILE = N_PAD // NS
CW = 16
ROW_BLK = 2000


def _sc_scatter(with_cnt):
    mesh = plsc.VectorSubcoreMesh(core_axis_name="c", subcore_axis_name="s")
    agg_t = jax.ShapeDtypeStruct((NC, N_PAD, D_HID), jnp.float32)
    out_type = [agg_t] if with_cnt else agg_t
    scratch = [
        pltpu.VMEM((EDGES_PER_TILE,), jnp.int32),
        pltpu.VMEM((EDGES_PER_TILE,), jnp.int32),
        pltpu.VMEM((CHUNK, D_HID), jnp.float32),
        pltpu.VMEM((CHUNK, D_HID), jnp.float32),
        pltpu.VMEM((TAIL, D_HID), jnp.float32),
        pltpu.VMEM_SHARED((N_PAD, D_HID), jnp.float32),
        pltpu.SemaphoreType.DMA,
        pltpu.SemaphoreType.DMA,
    ]
    if with_cnt:
        out_type.append(jax.ShapeDtypeStruct((NC, N_PAD, CW), jnp.float32))
        scratch += [
            pltpu.VMEM((CHUNK, CW), jnp.float32),
            pltpu.VMEM_SHARED((N_PAD, CW), jnp.float32),
            pltpu.SemaphoreType.DMA,
            pltpu.SemaphoreType.DMA,
        ]

    def body(*refs):
        if with_cnt:
            (p, ei, ones_h, z64, z16, agg_o, cnt_o,
             sv, dv, rA, rB, rT, acc, semA, semB,
             ones_v, cacc, semCA, semCB) = refs
        else:
            (p, ei, z64, agg_o,
             sv, dv, rA, rB, rT, acc, semA, semB) = refs

        cid = lax.axis_index("c")
        sid = lax.axis_index("s")
        wid = cid * NS + sid
        r0 = sid * ROWS_PER_TILE
        base = wid * EDGES_PER_TILE

        pltpu.sync_copy(z64, acc.at[pl.ds(r0, ROWS_PER_TILE)])
        if with_cnt:
            pltpu.sync_copy(z16, cacc.at[pl.ds(r0, ROWS_PER_TILE)])
            pltpu.sync_copy(ones_h, ones_v)
        pltpu.sync_copy(ei.at[0].at[pl.ds(base, EDGES_PER_TILE)], sv)
        pltpu.sync_copy(ei.at[1].at[pl.ds(base, EDGES_PER_TILE)], dv)
        plsc.subcore_barrier()

        def g_start(j, rv, sem):
            pltpu.async_copy(p.at[sv.at[pl.ds(j * CHUNK, CHUNK)]], rv, sem)

        def g_drain(j, rv, sem):
            pltpu.make_async_copy(
                p.at[sv.at[pl.ds(j * CHUNK, CHUNK)]], rv, sem).wait()

        def r_scat(j, rv):
            pltpu.sync_copy(rv, acc.at[dv.at[pl.ds(j * CHUNK, CHUNK)]],
                            add=True)

        def c_fire(j, sem):
            pltpu.async_copy(ones_v, cacc.at[dv.at[pl.ds(j * CHUNK, CHUNK)]],
                             sem, add=True)

        def c_wait(j, sem):
            pltpu.make_async_copy(
                ones_v, cacc.at[dv.at[pl.ds(j * CHUNK, CHUNK)]], sem).wait()

        tail_ds = pl.ds(FULL_CHUNKS * CHUNK, TAIL)

        g_start(0, rA, semA)
        if with_cnt:
            c_fire(0, semCA)
            c_fire(1, semCB)

        @pl.loop(0, FULL_CHUNKS // 2 - 1)
        def _(it):
            jA = it * 2
            jB = jA + 1
            g_start(jB, rB, semB)
            g_drain(jA, rA, semA)
            r_scat(jA, rA)
            if with_cnt:
                c_wait(jA, semCA)
                c_fire(jA + 2, semCA)
            g_start(jA + 2, rA, semA)
            g_drain(jB, rB, semB)
            r_scat(jB, rB)
            if with_cnt:
                c_wait(jB, semCB)
                c_fire(jB + 2, semCB)

        jA = FULL_CHUNKS - 2
        jB = FULL_CHUNKS - 1
        g_start(jB, rB, semB)
        g_drain(jA, rA, semA)
        r_scat(jA, rA)
        pltpu.async_copy(p.at[sv.at[tail_ds]], rT, semA)
        g_drain(jB, rB, semB)
        r_scat(jB, rB)
        pltpu.make_async_copy(p.at[sv.at[tail_ds]], rT, semA).wait()
        pltpu.sync_copy(rT, acc.at[dv.at[tail_ds]], add=True)
        if with_cnt:
            c_wait(jA, semCA)
            c_wait(jB, semCB)
            pltpu.sync_copy(ones_v.at[pl.ds(0, TAIL)],
                            cacc.at[dv.at[tail_ds]], add=True)

        plsc.subcore_barrier()
        pltpu.sync_copy(acc.at[pl.ds(r0, ROWS_PER_TILE)],
                        agg_o.at[cid].at[pl.ds(r0, ROWS_PER_TILE)])
        if with_cnt:
            pltpu.sync_copy(cacc.at[pl.ds(r0, ROWS_PER_TILE)],
                            cnt_o.at[cid].at[pl.ds(r0, ROWS_PER_TILE)])

    cp = pltpu.CompilerParams(use_tc_tiling_on_sc=False)
    return pl.kernel(body, out_type=out_type, mesh=mesh, scratch_types=scratch,
                     compiler_params=cp)


def _dense2(x, Wl, Wr, b2d):
    def tc_body(x_ref, wl_ref, wr_ref, b_ref, p_ref, q_ref):
        xb = x_ref[...]
        p_ref[...] = jnp.dot(xb, wl_ref[...],
                             preferred_element_type=jnp.float32,
                             precision=lax.Precision.HIGHEST)
        q_ref[...] = jnp.dot(xb, wr_ref[...],
                             preferred_element_type=jnp.float32,
                             precision=lax.Precision.HIGHEST) + b_ref[...]

    return pl.pallas_call(
        tc_body,
        grid=(N_NODES // ROW_BLK,),
        in_specs=[pl.BlockSpec((ROW_BLK, D_IN), lambda i: (i, 0)),
                  pl.BlockSpec((D_IN, D_HID), lambda i: (0, 0)),
                  pl.BlockSpec((D_IN, D_HID), lambda i: (0, 0)),
                  pl.BlockSpec((1, D_HID), lambda i: (0, 0))],
        out_specs=[pl.BlockSpec((ROW_BLK, D_HID), lambda i: (i, 0)),
                   pl.BlockSpec((ROW_BLK, D_HID), lambda i: (i, 0))],
        out_shape=[jax.ShapeDtypeStruct((N_NODES, D_HID), jnp.float32)] * 2,
    )(x, Wl, Wr, b2d)


def _mid(aggp, cntp, Q1, W2l, W2r, b2d):
    def tc_body(a_ref, c_ref, q_ref, wl_ref, wr_ref, b_ref, p_ref, q2_ref):
        a = a_ref[0] + a_ref[1]
        cnt = c_ref[0, :, 0:1] + c_ref[1, :, 0:1]
        inv = 1.0 / jnp.maximum(cnt, 1.0)
        h = jnp.maximum(a * inv + q_ref[...], 0.0)
        p_ref[...] = jnp.dot(h, wl_ref[...],
                             preferred_element_type=jnp.float32,
                             precision=lax.Precision.HIGHEST)
        q2_ref[...] = jnp.dot(h, wr_ref[...],
                              preferred_element_type=jnp.float32,
                              precision=lax.Precision.HIGHEST) + b_ref[...]

    return pl.pallas_call(
        tc_body,
        grid=(N_NODES // ROW_BLK,),
        in_specs=[pl.BlockSpec((NC, ROW_BLK, D_HID), lambda i: (0, i, 0)),
                  pl.BlockSpec((NC, ROW_BLK, CW), lambda i: (0, i, 0)),
                  pl.BlockSpec((ROW_BLK, D_HID), lambda i: (i, 0)),
                  pl.BlockSpec((D_HID, D_HID), lambda i: (0, 0)),
                  pl.BlockSpec((D_HID, D_HID), lambda i: (0, 0)),
                  pl.BlockSpec((1, D_HID), lambda i: (0, 0))],
        out_specs=[pl.BlockSpec((ROW_BLK, D_HID), lambda i: (i, 0)),
                   pl.BlockSpec((ROW_BLK, D_HID), lambda i: (i, 0))],
        out_shape=[jax.ShapeDtypeStruct((N_NODES, D_HID), jnp.float32)] * 2,
    )(aggp, cntp, Q1, W2l, W2r, b2d)


def _final(aggp, cntp, Q2, Wpad, bpad):
    def tc_body(a_ref, c_ref, q_ref, w_ref, b_ref, o_ref):
        a = a_ref[0] + a_ref[1]
        cnt = c_ref[0, :, 0:1] + c_ref[1, :, 0:1]
        inv = 1.0 / jnp.maximum(cnt, 1.0)
        h = jnp.maximum(a * inv + q_ref[...], 0.0)
        o_ref[...] = jnp.dot(h, w_ref[...],
                             preferred_element_type=jnp.float32,
                             precision=lax.Precision.HIGHEST) + b_ref[...]

    return pl.pallas_call(
        tc_body,
        grid=(N_NODES // ROW_BLK,),
        in_specs=[pl.BlockSpec((NC, ROW_BLK, D_HID), lambda i: (0, i, 0)),
                  pl.BlockSpec((NC, ROW_BLK, CW), lambda i: (0, i, 0)),
                  pl.BlockSpec((ROW_BLK, D_HID), lambda i: (i, 0)),
                  pl.BlockSpec((D_HID, 128), lambda i: (0, 0)),
                  pl.BlockSpec((1, 128), lambda i: (0, 0))],
        out_specs=pl.BlockSpec((ROW_BLK, 128), lambda i: (i, 0)),
        out_shape=jax.ShapeDtypeStruct((N_NODES, 128), jnp.float32),
    )(aggp, cntp, Q2, Wpad, bpad)


def kernel(x, edge_index, W1l, b1l, W1r, b1r, W2l, b2l, W2r, b2r, Wlin, blin):
    f32 = jnp.float32
    ei = edge_index.astype(jnp.int32)
    ones = jnp.ones((CHUNK, CW), f32)
    z64 = jnp.zeros((ROWS_PER_TILE, D_HID), f32)
    z16 = jnp.zeros((ROWS_PER_TILE, CW), f32)

    P1, Q1 = _dense2(x, W1l, W1r, (b1l + b1r).reshape(1, -1))
    agg1, cntp = _sc_scatter(True)(P1, ei, ones, z64, z16)
    P2, Q2 = _mid(agg1, cntp, Q1, W2l, W2r, (b2l + b2r).reshape(1, -1))
    agg2 = _sc_scatter(False)(P2, ei, z64)
    Wpad = jnp.pad(Wlin, ((0, 0), (0, 128 - D_OUT)))
    bpad = jnp.pad(blin, (0, 128 - D_OUT)).reshape(1, -1)
    outp = _final(agg2, cntp, Q2, Wpad, bpad)
    return outp[:, :D_OUT]
